# baseline (speedup 1.0000x reference)
import jax
import jax.numpy as jnp
import numpy as np
from jax.experimental import pallas as pl
from jax.experimental.pallas import tpu as pltpu

ROWS = 128
COLS = 128
GRID = 8
BLK = ROWS // GRID

THRESHOLD = np.float32(5e-8)
SCALE = np.float32(1e-7)


def _spl_loss_tc(x_ref, out_ref):
    i = pl.program_id(0)
    x = x_ref[...]
    keep = (x * SCALE) < THRESHOLD
    s = jnp.sum(jnp.where(keep, x, np.float32(0.0)))

    @pl.when(i == 0)
    def _():
        out_ref[0, 0] = s

    @pl.when(i > 0)
    def _():
        out_ref[0, 0] = out_ref[0, 0] + s


def kernel(super_loss, index, v):
    del index, v
    x2d = super_loss.reshape(ROWS, COLS)
    out = pl.pallas_call(
        _spl_loss_tc,
        grid=(GRID,),
        out_shape=jax.ShapeDtypeStruct((1, 1), jnp.float32),
        in_specs=[pl.BlockSpec((BLK, COLS), lambda i: (i, 0))],
        out_specs=pl.BlockSpec(memory_space=pltpu.SMEM),
    )(x2d)
    return out[0, 0]


# speedup vs baseline: 2.8528x; 2.8528x over previous
import jax
import jax.numpy as jnp
import numpy as np
from jax.experimental import pallas as pl
from jax.experimental.pallas import tpu as pltpu

ROWS = 128
COLS = 128

THRESHOLD = np.float32(5e-8)
SCALE = np.float32(1e-7)


def _spl_loss_tc(x_ref, out_ref):
    x = x_ref[...]
    keep = (x * SCALE) < THRESHOLD
    y = jnp.where(keep, x, np.float32(0.0))
    part = jnp.sum(y, axis=0, keepdims=True)
    out_ref[...] = jnp.sum(part, axis=1, keepdims=True)


def kernel(super_loss, index, v):
    del index, v
    x2d = super_loss.reshape(ROWS, COLS)
    out = pl.pallas_call(
        _spl_loss_tc,
        out_shape=jax.ShapeDtypeStruct((1, 1), jnp.float32),
    )(x2d)
    return out[0, 0]


# speedup vs baseline: 2.8817x; 1.0101x over previous
import jax
import jax.numpy as jnp
import numpy as np
from jax.experimental import pallas as pl

ROWS = 128
COLS = 128

THRESHOLD = np.float32(5e-8)
SCALE = np.float32(1e-7)


def _spl_loss_tc(x_ref, out_ref):
    x = x_ref[...]
    keep = (x * SCALE) < THRESHOLD
    y = jnp.where(keep, x, np.float32(0.0))
    part = jnp.sum(y, axis=0, keepdims=True)
    out_ref[...] = jnp.sum(part, axis=1, keepdims=True)


def kernel(super_loss, index, v):
    del index, v
    x2d = super_loss.reshape(ROWS, COLS)
    out = pl.pallas_call(
        _spl_loss_tc,
        out_shape=jax.ShapeDtypeStruct((1, 1), jnp.float32),
    )(x2d)
    return out[0, 0]
